# baseline (speedup 1.0000x reference)
import functools

import jax
import jax.numpy as jnp
from jax import lax
from jax.experimental import pallas as pl
from jax.experimental.pallas import tpu as pltpu
from jax.experimental.pallas import tpu_sc as plsc

N = 10000
E = 160000
D_IN = 256
HID = 128
HEADS = 4
D_OUT = 256
G = 64

NC = 2
NS = 16
LANES = 16
NW = NC * NS
EB = 64
PB = 128
NB_E = E // EB
N_PAD = 10240
NPT = N_PAD // NS
DEN_R = N_PAD // 128
BM = 1000


def _tc1_body(x_ref, w_ref, as_ref, ad_ref, t0, t1, t2, t3, aso, ado):
    xw = jnp.dot(x_ref[...], w_ref[...], preferred_element_type=jnp.float32)
    t0[...] = xw[:, 0:128]
    t1[...] = xw[:, 128:256]
    t2[...] = xw[:, 256:384]
    t3[...] = xw[:, 384:512]
    aso[...] = jnp.dot(xw, as_ref[...], preferred_element_type=jnp.float32)
    ado[...] = jnp.dot(xw, ad_ref[...], preferred_element_type=jnp.float32)


def _tc2_body(p0, p1, p2, p3, den_ref, b1_ref, w2_ref, as2_ref, ad2_ref,
              t20, t21, as2o, ad2o):
    d = den_ref[...] + 1e-16
    hs = []
    for h, p in enumerate((p0, p1, p2, p3)):
        hs.append(p[...] / d[:, h:h + 1])
    hcat = jnp.concatenate(hs, axis=1) + b1_ref[...]
    hcat = jnp.where(hcat > 0, hcat, jnp.exp(jnp.minimum(hcat, 0.0)) - 1.0)
    t2 = jnp.dot(hcat, w2_ref[...], preferred_element_type=jnp.float32)
    t20[...] = t2[:, 0:128]
    t21[...] = t2[:, 128:256]
    as2o[...] = jnp.dot(t2, as2_ref[...], preferred_element_type=jnp.float32)
    ad2o[...] = jnp.dot(t2, ad2_ref[...], preferred_element_type=jnp.float32)


def _tc3_body(q0, q1, den2_ref, b2_ref, h0_out, h1_out):
    d = den2_ref[...] + 1e-16
    b2v = b2_ref[...]
    h0_out[...] = q0[...] / d + b2v[:, 0:128]
    h1_out[...] = q1[...] / d + b2v[:, 128:256]


def _tc4_body(p0_ref, p1_ref, c_ref, out_ref):
    cnt = jnp.maximum((c_ref[0] + c_ref[1])[:, 0:1], 1.0)
    out_ref[...] = jnp.concatenate(
        [p0_ref[0] + p0_ref[1], p1_ref[0] + p1_ref[1]], axis=1) / cnt


def _sc_mesh(num_cores=NC):
    return plsc.VectorSubcoreMesh(core_axis_name="c", subcore_axis_name="s",
                                  num_cores=num_cores, num_subcores=NS)


def _make_edge_kernel():
    @functools.partial(
        pl.kernel,
        out_type=(jax.ShapeDtypeStruct((N_PAD, HID), jnp.float32),
                  jax.ShapeDtypeStruct((DEN_R, 128), jnp.float32)),
        mesh=_sc_mesh(1),
        compiler_params=pltpu.CompilerParams(needs_layout_passes=False),
        scratch_types=[
            pltpu.VMEM((N,), jnp.float32),
            pltpu.VMEM((N,), jnp.float32),
            pltpu.VMEM((DEN_R, 128), jnp.float32),
            pltpu.VMEM((DEN_R,), jnp.int32),
            pltpu.VMEM((EB,), jnp.int32),
            pltpu.VMEM((EB,), jnp.int32),
            pltpu.VMEM((EB,), jnp.float32),
            pltpu.VMEM((EB, HID), jnp.float32),
            pltpu.VMEM_SHARED((N_PAD, HID), jnp.float32),
            pltpu.VMEM_SHARED((DEN_R, 128), jnp.float32),
            pltpu.SemaphoreType.DMA,
        ],
    )
    def edge_kernel(asrc_hbm, adst_hbm, src_hbm, dst_hbm, table_hbm, rowid_hbm,
                    out_num, out_den,
                    asrc_v, adst_v, den_v, rid_v, src_v, dst_v, w_v, rows_v,
                    num_sp, den_sp, sem):
        s = lax.axis_index("s")
        wid = s
        nw = NS

        pltpu.sync_copy(asrc_hbm, asrc_v)
        pltpu.sync_copy(adst_hbm, adst_v)
        pltpu.sync_copy(rowid_hbm, rid_v)

        zero16 = jnp.zeros((LANES,), jnp.float32)

        def _zrow(r, _):
            for cc in range(HID // LANES):
                rows_v[r, pl.ds(cc * LANES, LANES)] = zero16
            return 0
        lax.fori_loop(0, EB, _zrow, 0)

        def _zden(r, _):
            for cc in range(128 // LANES):
                den_v[r, pl.ds(cc * LANES, LANES)] = zero16
            return 0
        lax.fori_loop(0, DEN_R, _zden, 0)

        nbase = s * NPT
        for k in range(NPT // EB):
            pltpu.sync_copy(rows_v, num_sp.at[pl.ds(nbase + k * EB, EB)])

        @pl.when(s < DEN_R // 8)
        def _():
            pltpu.sync_copy(den_v.at[pl.ds(0, 8)], den_sp.at[pl.ds(s * 8, 8)])
        plsc.subcore_barrier()

        nfull = NB_E // NS
        nb = jnp.where(wid < NB_E - nfull * NS, nfull + 1, nfull)

        def batch_body(j, _):
            base = (wid + j * nw) * EB
            pltpu.sync_copy(src_hbm.at[pl.ds(base, EB)], src_v)
            pltpu.sync_copy(dst_hbm.at[pl.ds(base, EB)], dst_v)
            cp = pltpu.async_copy(table_hbm.at[src_v], rows_v, sem)
            for t in range(EB // LANES):
                s16 = src_v[pl.ds(t * LANES, LANES)]
                d16 = dst_v[pl.ds(t * LANES, LANES)]
                e = plsc.load_gather(asrc_v, [s16]) + plsc.load_gather(adst_v, [d16])
                w = jnp.exp(jnp.maximum(e, 0.2 * e))
                w_v[pl.ds(t * LANES, LANES)] = w
                plsc.addupdate_scatter(
                    den_v,
                    [lax.shift_right_logical(d16, 7), lax.bitwise_and(d16, 127)],
                    w)
            cp.wait()

            def row_body(r, _):
                wb = plsc.load_gather(w_v, [jnp.zeros((LANES,), jnp.int32) + r])
                for cc in range(HID // LANES):
                    sl = pl.ds(cc * LANES, LANES)
                    rows_v[r, sl] = rows_v[r, sl] * wb
                return 0
            lax.fori_loop(0, EB, row_body, 0)
            pltpu.sync_copy(rows_v, num_sp.at[dst_v], add=True)
            return 0
        lax.fori_loop(0, nb, batch_body, 0)

        pltpu.sync_copy(den_v, den_sp.at[rid_v], add=True)
        plsc.subcore_barrier()

        for k in range(NPT // EB):
            pltpu.sync_copy(num_sp.at[pl.ds(nbase + k * EB, EB)],
                            out_num.at[pl.ds(nbase + k * EB, EB)])

        @pl.when(s < DEN_R // 8)
        def _():
            pltpu.sync_copy(den_sp.at[pl.ds(s * 8, 8)],
                            out_den.at[pl.ds(s * 8, 8)])

    return edge_kernel


def _make_pool_kernel():
    NBP = N // PB
    TAIL = N - NBP * PB

    @functools.partial(
        pl.kernel,
        out_type=(jax.ShapeDtypeStruct((NC, G, 128), jnp.float32),
                  jax.ShapeDtypeStruct((NC, G, 128), jnp.float32),
                  jax.ShapeDtypeStruct((NC, G, 128), jnp.float32)),
        mesh=_sc_mesh(),
        compiler_params=pltpu.CompilerParams(needs_layout_passes=False),
        scratch_types=[
            pltpu.VMEM((PB, 128), jnp.float32),
            pltpu.VMEM((PB, 128), jnp.float32),
            pltpu.VMEM((PB,), jnp.int32),
            pltpu.VMEM((TAIL, 128), jnp.float32),
            pltpu.VMEM((TAIL, 128), jnp.float32),
            pltpu.VMEM((TAIL,), jnp.int32),
            pltpu.VMEM((G, 128), jnp.float32),
            pltpu.VMEM((G,), jnp.int32),
            pltpu.VMEM_SHARED((G, 128), jnp.float32),
            pltpu.VMEM_SHARED((G, 128), jnp.float32),
            pltpu.VMEM_SHARED((G, 128), jnp.float32),
        ],
    )
    def pool_kernel(h0_hbm, h1_hbm, batch_hbm, gid_hbm,
                    out_p0, out_p1, out_cnt,
                    rows0_v, rows1_v, bid_v, trows0_v, trows1_v, tbid_v,
                    cnt_v, rid_v, p0_sp, p1_sp, cnt_sp):
        c = lax.axis_index("c")
        s = lax.axis_index("s")
        wid = c * NS + s

        pltpu.sync_copy(gid_hbm, rid_v)
        zero16 = jnp.zeros((LANES,), jnp.float32)
        one16 = jnp.ones((LANES,), jnp.float32)

        def _zcnt(r, _):
            for cc in range(128 // LANES):
                cnt_v[r, pl.ds(cc * LANES, LANES)] = zero16
                rows0_v[r, pl.ds(cc * LANES, LANES)] = zero16
            return 0
        lax.fori_loop(0, G, _zcnt, 0)

        @pl.when(s < G // 8)
        def _():
            pltpu.sync_copy(rows0_v.at[pl.ds(0, 8)], p0_sp.at[pl.ds(s * 8, 8)])
            pltpu.sync_copy(rows0_v.at[pl.ds(0, 8)], p1_sp.at[pl.ds(s * 8, 8)])
            pltpu.sync_copy(cnt_v.at[pl.ds(0, 8)], cnt_sp.at[pl.ds(s * 8, 8)])
        plsc.subcore_barrier()

        nfull = NBP // NW
        nb = jnp.where(wid < NBP - nfull * NW, nfull + 1, nfull)

        def pb(j, _):
            base = (wid + j * NW) * PB
            pltpu.sync_copy(h0_hbm.at[pl.ds(base, PB)], rows0_v)
            pltpu.sync_copy(h1_hbm.at[pl.ds(base, PB)], rows1_v)
            pltpu.sync_copy(batch_hbm.at[pl.ds(base, PB)], bid_v)
            for t in range(PB // LANES):
                b16 = bid_v[pl.ds(t * LANES, LANES)]
                plsc.addupdate_scatter(
                    cnt_v, [b16, jnp.zeros((LANES,), jnp.int32)], one16)
            pltpu.sync_copy(rows0_v, p0_sp.at[bid_v], add=True)
            pltpu.sync_copy(rows1_v, p1_sp.at[bid_v], add=True)
            return 0
        lax.fori_loop(0, nb, pb, 0)

        @pl.when(wid == NW - 1)
        def _():
            pltpu.sync_copy(h0_hbm.at[pl.ds(N - TAIL, TAIL)], trows0_v)
            pltpu.sync_copy(h1_hbm.at[pl.ds(N - TAIL, TAIL)], trows1_v)
            pltpu.sync_copy(batch_hbm.at[pl.ds(N - TAIL, TAIL)], tbid_v)
            t16 = tbid_v[pl.ds(0, LANES)]
            plsc.addupdate_scatter(
                cnt_v, [t16, jnp.zeros((LANES,), jnp.int32)], one16)
            pltpu.sync_copy(trows0_v, p0_sp.at[tbid_v], add=True)
            pltpu.sync_copy(trows1_v, p1_sp.at[tbid_v], add=True)

        pltpu.sync_copy(cnt_v, cnt_sp.at[rid_v], add=True)
        plsc.subcore_barrier()

        @pl.when(s < G // 8)
        def _():
            pltpu.sync_copy(p0_sp.at[pl.ds(s * 8, 8)],
                            out_p0.at[c, pl.ds(s * 8, 8)])
            pltpu.sync_copy(p1_sp.at[pl.ds(s * 8, 8)],
                            out_p1.at[c, pl.ds(s * 8, 8)])
            pltpu.sync_copy(cnt_sp.at[pl.ds(s * 8, 8)],
                            out_cnt.at[c, pl.ds(s * 8, 8)])

    return pool_kernel


def kernel(x, edge_index, batch, W1, att_src1, att_dst1, b1,
           W2, att_src2, att_dst2, b2):
    x = x.astype(jnp.float32)
    src = edge_index[0]
    dst = edge_index[1]
    rowids = jnp.arange(DEN_R, dtype=jnp.int32)

    eyeH = jnp.eye(HEADS, dtype=jnp.float32)
    As1 = jnp.einsum("hd,hg->hdg", att_src1, eyeH).reshape(HEADS * HID, HEADS)
    Ad1 = jnp.einsum("hd,hg->hdg", att_dst1, eyeH).reshape(HEADS * HID, HEADS)
    As2 = att_src2.reshape(D_OUT, 1)
    Ad2 = att_dst2.reshape(D_OUT, 1)

    grid = (N // BM,)
    row_spec = lambda w: pl.BlockSpec((BM, w), lambda i: (i, 0))
    full_spec = lambda a, b_: pl.BlockSpec((a, b_), lambda i: (0, 0))
    t10, t11, t12, t13, as1, ad1 = pl.pallas_call(
        _tc1_body,
        grid=grid,
        in_specs=[row_spec(D_IN), full_spec(D_IN, HEADS * HID),
                  full_spec(HEADS * HID, HEADS), full_spec(HEADS * HID, HEADS)],
        out_specs=[row_spec(HID)] * 4 + [row_spec(HEADS)] * 2,
        out_shape=[jax.ShapeDtypeStruct((N, HID), jnp.float32)] * 4
                  + [jax.ShapeDtypeStruct((N, HEADS), jnp.float32)] * 2,
    )(x, W1, As1, Ad1)

    edge_k = _make_edge_kernel()
    tables1 = (t10, t11, t12, t13)
    nums1, dens1 = [], []
    for h in range(HEADS):
        on, od = edge_k(as1[:, h], ad1[:, h], src, dst, tables1[h], rowids)
        nums1.append(on)
        dens1.append(od)
    den1 = jnp.stack([p.reshape(N_PAD) for p in dens1], axis=-1)

    t20, t21, as2, ad2 = pl.pallas_call(
        _tc2_body,
        grid=grid,
        in_specs=[row_spec(HID)] * 4
                 + [pl.BlockSpec((BM, HEADS), lambda i: (i, 0)),
                    full_spec(1, HEADS * HID),
                    full_spec(HEADS * HID, D_OUT),
                    full_spec(D_OUT, 1), full_spec(D_OUT, 1)],
        out_specs=[row_spec(HID)] * 2 + [row_spec(1)] * 2,
        out_shape=[jax.ShapeDtypeStruct((N, HID), jnp.float32)] * 2
                  + [jax.ShapeDtypeStruct((N, 1), jnp.float32)] * 2,
    )(nums1[0], nums1[1], nums1[2], nums1[3], den1, b1.reshape(1, HEADS * HID),
      W2, As2, Ad2)

    q0n, q0d = edge_k(as2[:, 0], ad2[:, 0], src, dst, t20, rowids)
    q1n, _ = edge_k(as2[:, 0], ad2[:, 0], src, dst, t21, rowids)
    den2 = q0d.reshape(N_PAD)[:, None]

    h0, h1 = pl.pallas_call(
        _tc3_body,
        grid=grid,
        in_specs=[row_spec(HID), row_spec(HID),
                  pl.BlockSpec((BM, 1), lambda i: (i, 0)),
                  full_spec(1, D_OUT)],
        out_specs=[row_spec(HID)] * 2,
        out_shape=[jax.ShapeDtypeStruct((N, HID), jnp.float32)] * 2,
    )(q0n, q1n, den2, b2.reshape(1, D_OUT))

    pool_k = _make_pool_kernel()
    gids = jnp.arange(G, dtype=jnp.int32)
    pool_p0, pool_p1, cnt_part = pool_k(h0, h1, batch, gids)

    pooled = pl.pallas_call(
        _tc4_body,
        grid=(1,),
        in_specs=[pl.BlockSpec((NC, G, 128), lambda i: (0, 0, 0))] * 3,
        out_specs=pl.BlockSpec((G, D_OUT), lambda i: (0, 0)),
        out_shape=jax.ShapeDtypeStruct((G, D_OUT), jnp.float32),
    )(pool_p0, pool_p1, cnt_part)
    return pooled


# speedup vs baseline: 4.0926x; 4.0926x over previous
import functools

import jax
import jax.numpy as jnp
from jax import lax
from jax.experimental import pallas as pl
from jax.experimental.pallas import tpu as pltpu
from jax.experimental.pallas import tpu_sc as plsc

N = 10000
E = 160000
D_IN = 256
HID = 128
HEADS = 4
D_OUT = 256
G = 64

NC = 2
NS = 16
LANES = 16
NW = NC * NS
N_PAD = 10240
NPT = N_PAD // NS
DEN_R = N_PAD // 128
EB = 64
ECH = 640
NBCH = ECH // EB
NCHG = E // ECH
NCHF = NCHG // NS
NCHR = NCHG - NCHF * NS
PB = 128
BM = 1000


def _tc1_body(x_ref, w_ref, as_ref, ad_ref, t01, t23, aso, ado):
    xw = jnp.dot(x_ref[...], w_ref[...], preferred_element_type=jnp.float32)
    t01[0] = xw[:, 0:128]
    t01[1] = xw[:, 128:256]
    t23[0] = xw[:, 256:384]
    t23[1] = xw[:, 384:512]
    aso[...] = jnp.dot(xw, as_ref[...], preferred_element_type=jnp.float32)
    ado[...] = jnp.dot(xw, ad_ref[...], preferred_element_type=jnp.float32)


def _tc2_body(p0, p1, p2, p3, den_ref, b1_ref, w2_ref, as2_ref, ad2_ref,
              t2p, as2o, ad2o):
    d = den_ref[...] + 1e-16
    hs = []
    for h, p in enumerate((p0, p1, p2, p3)):
        hs.append(p[0] / d[:, h:h + 1])
    hcat = jnp.concatenate(hs, axis=1) + b1_ref[...]
    hcat = jnp.where(hcat > 0, hcat, jnp.exp(jnp.minimum(hcat, 0.0)) - 1.0)
    t2 = jnp.dot(hcat, w2_ref[...], preferred_element_type=jnp.float32)
    t2p[0] = t2[:, 0:128]
    t2p[1] = t2[:, 128:256]
    as2o[...] = jnp.dot(t2, as2_ref[...], preferred_element_type=jnp.float32)
    ad2o[...] = jnp.dot(t2, ad2_ref[...], preferred_element_type=jnp.float32)


def _tc3_body(q0, q1, den2_ref, b2_ref, h0_out, h1_out):
    d = den2_ref[...] + 1e-16
    b2v = b2_ref[...]
    h0_out[...] = q0[0] / d + b2v[:, 0:128]
    h1_out[...] = q1[0] / d + b2v[:, 128:256]


def _tc4_body(p0_ref, p1_ref, c_ref, out_ref):
    cnt = jnp.maximum((c_ref[0] + c_ref[1])[:, 0:1], 1.0)
    out_ref[...] = jnp.concatenate(
        [p0_ref[0] + p0_ref[1], p1_ref[0] + p1_ref[1]], axis=1) / cnt


def _sc_mesh():
    return plsc.VectorSubcoreMesh(core_axis_name="c", subcore_axis_name="s",
                                  num_cores=NC, num_subcores=NS)


def _lrelu_exp(e):
    return jnp.exp(jnp.maximum(e, 0.2 * e))


def _make_w_kernel(num_heads):
    hpc = 2 if num_heads == 4 else 1

    @functools.partial(
        pl.kernel,
        out_type=jax.ShapeDtypeStruct((num_heads, 1, E), jnp.float32),
        mesh=_sc_mesh(),
        compiler_params=pltpu.CompilerParams(needs_layout_passes=False),
        scratch_types=[pltpu.VMEM((N,), jnp.float32)] * (2 * hpc)
                      + [pltpu.VMEM((ECH,), jnp.int32)] * 2
                      + [pltpu.VMEM((ECH,), jnp.float32)] * hpc
                      + [pltpu.SemaphoreType.DMA],
    )
    def w_kernel(asT_h, adT_h, src_h, dst_h, w_out, *refs):
        a_vs = refs[:hpc]
        b_vs = refs[hpc:2 * hpc]
        ssrc, sdst = refs[2 * hpc:2 * hpc + 2]
        wbufs = refs[2 * hpc + 2:2 * hpc + 2 + hpc]
        sem = refs[-1]
        c = lax.axis_index("c")
        s = lax.axis_index("s")

        def tile_work():
            for hi in range(hpc):
                pltpu.sync_copy(asT_h.at[hpc * c + hi, 0], a_vs[hi])
                pltpu.sync_copy(adT_h.at[hpc * c + hi, 0], b_vs[hi])
            nch = jnp.where(s < NCHR, NCHF + 1, NCHF)

            def chunk(k, _):
                cb = (s + k * NS) * ECH
                c1 = pltpu.async_copy(src_h.at[pl.ds(cb, ECH)], ssrc, sem)
                c2 = pltpu.async_copy(dst_h.at[pl.ds(cb, ECH)], sdst, sem)
                c1.wait()
                c2.wait()
                for t in range(ECH // LANES):
                    sl = pl.ds(t * LANES, LANES)
                    s16 = ssrc[sl]
                    d16 = sdst[sl]
                    for hi in range(hpc):
                        e = (plsc.load_gather(a_vs[hi], [s16])
                             + plsc.load_gather(b_vs[hi], [d16]))
                        wbufs[hi][sl] = _lrelu_exp(e)
                for hi in range(hpc):
                    pltpu.sync_copy(wbufs[hi],
                                    w_out.at[hpc * c + hi, 0, pl.ds(cb, ECH)])
                return 0
            lax.fori_loop(0, nch, chunk, 0)

        if num_heads == 4:
            tile_work()
        else:
            @pl.when(c == 0)
            def _():
                tile_work()

    return w_kernel


def _make_pair_kernel():
    @functools.partial(
        pl.kernel,
        out_type=(jax.ShapeDtypeStruct((NC, N_PAD, HID), jnp.float32),
                  jax.ShapeDtypeStruct((NC, DEN_R, 128), jnp.float32)),
        mesh=_sc_mesh(),
        compiler_params=pltpu.CompilerParams(needs_layout_passes=False),
        scratch_types=[
            pltpu.VMEM((DEN_R, 128), jnp.float32),
            pltpu.VMEM((DEN_R,), jnp.int32),
            pltpu.VMEM((ECH,), jnp.int32),
            pltpu.VMEM((ECH,), jnp.int32),
            pltpu.VMEM((ECH,), jnp.float32),
        ]
        + [pltpu.VMEM((EB, HID), jnp.float32)] * 4
        + [pltpu.VMEM((EB,), jnp.int32)] * 4
        + [pltpu.VMEM_SHARED((N_PAD, HID), jnp.float32),
           pltpu.VMEM_SHARED((DEN_R, 128), jnp.float32)]
        + [pltpu.SemaphoreType.DMA] * 9,
    )
    def pair_kernel(src_h, dst_h, w_h, tbl_h, rid_h, out_num, out_den,
                    den_v, rid_v, ssrc, sdst, sw,
                    r0, r1, r2, r3, db0, db1, db2, db3,
                    num_sp, den_sp,
                    sem_st, g0, g1, g2, g3, s0, s1, s2, s3):
        rows = (r0, r1, r2, r3)
        dbs = (db0, db1, db2, db3)
        gsems = (g0, g1, g2, g3)
        ssems = (s0, s1, s2, s3)
        c = lax.axis_index("c")
        s = lax.axis_index("s")
        wline = w_h.at[c, 0]
        tline = tbl_h.at[c]

        pltpu.sync_copy(rid_h, rid_v)
        zero16 = jnp.zeros((LANES,), jnp.float32)

        def _zrow(r, _):
            for cc in range(HID // LANES):
                r0[r, pl.ds(cc * LANES, LANES)] = zero16
            return 0
        lax.fori_loop(0, EB, _zrow, 0)

        def _zden(r, _):
            for cc in range(128 // LANES):
                den_v[r, pl.ds(cc * LANES, LANES)] = zero16
            return 0
        lax.fori_loop(0, DEN_R, _zden, 0)

        nbase = s * NPT
        for k in range(NPT // EB):
            pltpu.sync_copy(r0, num_sp.at[pl.ds(nbase + k * EB, EB)])

        @pl.when(s < DEN_R // 8)
        def _():
            pltpu.sync_copy(den_v.at[pl.ds(0, 8)], den_sp.at[pl.ds(s * 8, 8)])
        plsc.subcore_barrier()

        def _den_update(idx_ref, woff, n16):
            for t in range(n16):
                sl = pl.ds(t * LANES, LANES)
                d16 = idx_ref[sl]
                plsc.addupdate_scatter(
                    den_v,
                    [lax.shift_right_logical(d16, 7),
                     lax.bitwise_and(d16, 127)],
                    sw[pl.ds(woff + t * LANES, LANES)])

        def _scale_rows(buf, woff, nrows):
            def body(r, _):
                wb = plsc.load_gather(sw, [jnp.zeros((LANES,), jnp.int32)
                                           + (woff + r)])
                for cc in range(HID // LANES):
                    sl = pl.ds(cc * LANES, LANES)
                    buf[r, sl] = buf[r, sl] * wb
                return 0
            lax.fori_loop(0, nrows, body, 0)

        nch = jnp.where(s < NCHR, NCHF + 1, NCHF)

        def chunk(k, _):
            cb = (s + k * NS) * ECH
            c1 = pltpu.async_copy(src_h.at[pl.ds(cb, ECH)], ssrc, sem_st)
            c2 = pltpu.async_copy(dst_h.at[pl.ds(cb, ECH)], sdst, sem_st)
            c3 = pltpu.async_copy(wline.at[pl.ds(cb, ECH)], sw, sem_st)
            c1.wait()
            c2.wait()
            c3.wait()
            gdesc = {}
            sdesc = {}
            for b in (0, 1):
                gdesc[b] = pltpu.async_copy(
                    tline.at[ssrc.at[pl.ds(b * EB, EB)]], rows[b], gsems[b])
            for b in range(NBCH):
                nb = b + 2
                if nb < NBCH:
                    if nb - 4 >= 0:
                        sdesc[nb - 4].wait()
                    gdesc[nb] = pltpu.async_copy(
                        tline.at[ssrc.at[pl.ds(nb * EB, EB)]],
                        rows[nb % 4], gsems[nb % 4])
                gdesc[b].wait()
                for t in range(EB // LANES):
                    sl = pl.ds(t * LANES, LANES)
                    dbs[b % 4][sl] = sdst[pl.ds(b * EB + t * LANES, LANES)]
                _den_update(dbs[b % 4], b * EB, EB // LANES)
                _scale_rows(rows[b % 4], b * EB, EB)
                sdesc[b] = pltpu.async_copy(rows[b % 4],
                                            num_sp.at[dbs[b % 4]],
                                            ssems[b % 4], add=True)
            for b in range(NBCH - 4, NBCH):
                sdesc[b].wait()
            return 0
        lax.fori_loop(0, nch, chunk, 0)

        pltpu.sync_copy(den_v, den_sp.at[rid_v], add=True)
        plsc.subcore_barrier()

        for k in range(NPT // EB):
            pltpu.sync_copy(num_sp.at[pl.ds(nbase + k * EB, EB)],
                            out_num.at[c, pl.ds(nbase + k * EB, EB)])

        @pl.when(s < DEN_R // 8)
        def _():
            pltpu.sync_copy(den_sp.at[pl.ds(s * 8, 8)],
                            out_den.at[c, pl.ds(s * 8, 8)])

    return pair_kernel


def _make_pool_kernel():
    NBP = N // PB
    TAIL = N - NBP * PB

    @functools.partial(
        pl.kernel,
        out_type=(jax.ShapeDtypeStruct((NC, G, 128), jnp.float32),
                  jax.ShapeDtypeStruct((NC, G, 128), jnp.float32),
                  jax.ShapeDtypeStruct((NC, G, 128), jnp.float32)),
        mesh=_sc_mesh(),
        compiler_params=pltpu.CompilerParams(needs_layout_passes=False),
        scratch_types=[
            pltpu.VMEM((PB, 128), jnp.float32),
            pltpu.VMEM((PB, 128), jnp.float32),
            pltpu.VMEM((PB,), jnp.int32),
            pltpu.VMEM((TAIL, 128), jnp.float32),
            pltpu.VMEM((TAIL, 128), jnp.float32),
            pltpu.VMEM((TAIL,), jnp.int32),
            pltpu.VMEM((G, 128), jnp.float32),
            pltpu.VMEM((G,), jnp.int32),
            pltpu.VMEM_SHARED((G, 128), jnp.float32),
            pltpu.VMEM_SHARED((G, 128), jnp.float32),
            pltpu.VMEM_SHARED((G, 128), jnp.float32),
        ],
    )
    def pool_kernel(h0_hbm, h1_hbm, batch_hbm, gid_hbm,
                    out_p0, out_p1, out_cnt,
                    rows0_v, rows1_v, bid_v, trows0_v, trows1_v, tbid_v,
                    cnt_v, rid_v, p0_sp, p1_sp, cnt_sp):
        c = lax.axis_index("c")
        s = lax.axis_index("s")
        wid = c * NS + s

        pltpu.sync_copy(gid_hbm, rid_v)
        zero16 = jnp.zeros((LANES,), jnp.float32)
        one16 = jnp.ones((LANES,), jnp.float32)

        def _zcnt(r, _):
            for cc in range(128 // LANES):
                cnt_v[r, pl.ds(cc * LANES, LANES)] = zero16
                rows0_v[r, pl.ds(cc * LANES, LANES)] = zero16
            return 0
        lax.fori_loop(0, G, _zcnt, 0)

        @pl.when(s < G // 8)
        def _():
            pltpu.sync_copy(rows0_v.at[pl.ds(0, 8)], p0_sp.at[pl.ds(s * 8, 8)])
            pltpu.sync_copy(rows0_v.at[pl.ds(0, 8)], p1_sp.at[pl.ds(s * 8, 8)])
            pltpu.sync_copy(cnt_v.at[pl.ds(0, 8)], cnt_sp.at[pl.ds(s * 8, 8)])
        plsc.subcore_barrier()

        nfull = NBP // NW
        nb = jnp.where(wid < NBP - nfull * NW, nfull + 1, nfull)

        def pb(j, _):
            base = (wid + j * NW) * PB
            pltpu.sync_copy(h0_hbm.at[pl.ds(base, PB)], rows0_v)
            pltpu.sync_copy(h1_hbm.at[pl.ds(base, PB)], rows1_v)
            pltpu.sync_copy(batch_hbm.at[pl.ds(base, PB)], bid_v)
            for t in range(PB // LANES):
                b16 = bid_v[pl.ds(t * LANES, LANES)]
                plsc.addupdate_scatter(
                    cnt_v, [b16, jnp.zeros((LANES,), jnp.int32)], one16)
            pltpu.sync_copy(rows0_v, p0_sp.at[bid_v], add=True)
            pltpu.sync_copy(rows1_v, p1_sp.at[bid_v], add=True)
            return 0
        lax.fori_loop(0, nb, pb, 0)

        @pl.when(wid == NW - 1)
        def _():
            pltpu.sync_copy(h0_hbm.at[pl.ds(N - TAIL, TAIL)], trows0_v)
            pltpu.sync_copy(h1_hbm.at[pl.ds(N - TAIL, TAIL)], trows1_v)
            pltpu.sync_copy(batch_hbm.at[pl.ds(N - TAIL, TAIL)], tbid_v)
            t16 = tbid_v[pl.ds(0, LANES)]
            plsc.addupdate_scatter(
                cnt_v, [t16, jnp.zeros((LANES,), jnp.int32)], one16)
            pltpu.sync_copy(trows0_v, p0_sp.at[tbid_v], add=True)
            pltpu.sync_copy(trows1_v, p1_sp.at[tbid_v], add=True)

        pltpu.sync_copy(cnt_v, cnt_sp.at[rid_v], add=True)
        plsc.subcore_barrier()

        @pl.when(s < G // 8)
        def _():
            pltpu.sync_copy(p0_sp.at[pl.ds(s * 8, 8)],
                            out_p0.at[c, pl.ds(s * 8, 8)])
            pltpu.sync_copy(p1_sp.at[pl.ds(s * 8, 8)],
                            out_p1.at[c, pl.ds(s * 8, 8)])
            pltpu.sync_copy(cnt_sp.at[pl.ds(s * 8, 8)],
                            out_cnt.at[c, pl.ds(s * 8, 8)])

    return pool_kernel


def kernel(x, edge_index, batch, W1, att_src1, att_dst1, b1,
           W2, att_src2, att_dst2, b2):
    x = x.astype(jnp.float32)
    src = edge_index[0]
    dst = edge_index[1]
    rowids = jnp.arange(DEN_R, dtype=jnp.int32)

    eyeH = jnp.eye(HEADS, dtype=jnp.float32)
    As1 = jnp.einsum("hd,hg->hdg", att_src1, eyeH).reshape(HEADS * HID, HEADS)
    Ad1 = jnp.einsum("hd,hg->hdg", att_dst1, eyeH).reshape(HEADS * HID, HEADS)
    As2 = att_src2.reshape(D_OUT, 1)
    Ad2 = att_dst2.reshape(D_OUT, 1)

    grid = (N // BM,)
    row_spec = lambda w: pl.BlockSpec((BM, w), lambda i: (i, 0))
    pair_spec = pl.BlockSpec((NC, BM, HID), lambda i: (0, i, 0))
    full_spec = lambda a, b_: pl.BlockSpec((a, b_), lambda i: (0, 0))
    t01, t23, as1, ad1 = pl.pallas_call(
        _tc1_body,
        grid=grid,
        in_specs=[row_spec(D_IN), full_spec(D_IN, HEADS * HID),
                  full_spec(HEADS * HID, HEADS), full_spec(HEADS * HID, HEADS)],
        out_specs=[pair_spec] * 2 + [row_spec(HEADS)] * 2,
        out_shape=[jax.ShapeDtypeStruct((NC, N, HID), jnp.float32)] * 2
                  + [jax.ShapeDtypeStruct((N, HEADS), jnp.float32)] * 2,
    )(x, W1, As1, Ad1)

    w_k4 = _make_w_kernel(HEADS)
    w1 = w_k4(as1.T.reshape(HEADS, 1, N), ad1.T.reshape(HEADS, 1, N),
              src, dst)

    pair_k = _make_pair_kernel()
    n01, d01 = pair_k(src, dst, w1[0:2], t01, rowids)
    n23, d23 = pair_k(src, dst, w1[2:4], t23, rowids)
    den1 = jnp.stack([d01[0].reshape(N_PAD), d01[1].reshape(N_PAD),
                      d23[0].reshape(N_PAD), d23[1].reshape(N_PAD)], axis=-1)

    plane0 = pl.BlockSpec((1, BM, HID), lambda i: (0, i, 0))
    plane1 = pl.BlockSpec((1, BM, HID), lambda i: (1, i, 0))
    t2p, as2, ad2 = pl.pallas_call(
        _tc2_body,
        grid=grid,
        in_specs=[plane0, plane1, plane0, plane1,
                  pl.BlockSpec((BM, HEADS), lambda i: (i, 0)),
                  full_spec(1, HEADS * HID),
                  full_spec(HEADS * HID, D_OUT),
                  full_spec(D_OUT, 1), full_spec(D_OUT, 1)],
        out_specs=[pair_spec, row_spec(1), row_spec(1)],
        out_shape=[jax.ShapeDtypeStruct((NC, N, HID), jnp.float32),
                   jax.ShapeDtypeStruct((N, 1), jnp.float32),
                   jax.ShapeDtypeStruct((N, 1), jnp.float32)],
    )(n01, n01, n23, n23, den1, b1.reshape(1, HEADS * HID), W2, As2, Ad2)

    w_k1 = _make_w_kernel(1)
    w2 = w_k1(as2.reshape(1, 1, N), ad2.reshape(1, 1, N), src, dst)
    w2p = jnp.concatenate([w2, w2], axis=0)

    q01, dq = pair_k(src, dst, w2p, t2p, rowids)
    den2 = dq[0].reshape(N_PAD)[:, None]

    h0, h1 = pl.pallas_call(
        _tc3_body,
        grid=grid,
        in_specs=[plane0, plane1,
                  pl.BlockSpec((BM, 1), lambda i: (i, 0)),
                  full_spec(1, D_OUT)],
        out_specs=[row_spec(HID)] * 2,
        out_shape=[jax.ShapeDtypeStruct((N, HID), jnp.float32)] * 2,
    )(q01, q01, den2, b2.reshape(1, D_OUT))

    pool_k = _make_pool_kernel()
    gids = jnp.arange(G, dtype=jnp.int32)
    pool_p0, pool_p1, cnt_part = pool_k(h0, h1, batch, gids)

    pooled = pl.pallas_call(
        _tc4_body,
        grid=(1,),
        in_specs=[pl.BlockSpec((NC, G, 128), lambda i: (0, 0, 0))] * 3,
        out_specs=pl.BlockSpec((G, D_OUT), lambda i: (0, 0)),
        out_shape=jax.ShapeDtypeStruct((G, D_OUT), jnp.float32),
    )(pool_p0, pool_p1, cnt_part)
    return pooled


# speedup vs baseline: 4.4062x; 1.0766x over previous
import functools

import jax
import jax.numpy as jnp
from jax import lax
from jax.experimental import pallas as pl
from jax.experimental.pallas import tpu as pltpu
from jax.experimental.pallas import tpu_sc as plsc

N = 10000
E = 160000
D_IN = 256
HID = 128
HEADS = 4
D_OUT = 256
G = 64

NC = 2
NS = 16
LANES = 16
NW = NC * NS
N_PAD = 10240
NPT = N_PAD // NS
DEN_R = N_PAD // 128
EB = 64
ECH = 640
NBCH = ECH // EB
NCHG = E // ECH
NCHF = NCHG // NS
NCHR = NCHG - NCHF * NS
PB = 128
BM = 1000


def _tc1_body(x_ref, w_ref, as_ref, ad_ref, t01, t23, aso, ado):
    xw = jnp.dot(x_ref[...], w_ref[...], preferred_element_type=jnp.float32)
    t01[0] = xw[:, 0:128]
    t01[1] = xw[:, 128:256]
    t23[0] = xw[:, 256:384]
    t23[1] = xw[:, 384:512]
    aso[...] = jnp.dot(xw, as_ref[...], preferred_element_type=jnp.float32)
    ado[...] = jnp.dot(xw, ad_ref[...], preferred_element_type=jnp.float32)


def _tc2_body(p0, p1, p2, p3, den_ref, b1_ref, w2_ref, as2_ref, ad2_ref,
              t2p, as2o, ad2o):
    d = den_ref[...] + 1e-16
    hs = []
    for h, p in enumerate((p0, p1, p2, p3)):
        hs.append(p[0] / d[:, h:h + 1])
    hcat = jnp.concatenate(hs, axis=1) + b1_ref[...]
    hcat = jnp.where(hcat > 0, hcat, jnp.exp(jnp.minimum(hcat, 0.0)) - 1.0)
    t2 = jnp.dot(hcat, w2_ref[...], preferred_element_type=jnp.float32)
    t2p[0] = t2[:, 0:128]
    t2p[1] = t2[:, 128:256]
    as2o[...] = jnp.dot(t2, as2_ref[...], preferred_element_type=jnp.float32)
    ad2o[...] = jnp.dot(t2, ad2_ref[...], preferred_element_type=jnp.float32)


def _tc3_body(q0, q1, den2_ref, b2_ref, h0_out, h1_out):
    d = den2_ref[...] + 1e-16
    b2v = b2_ref[...]
    h0_out[...] = q0[0] / d + b2v[:, 0:128]
    h1_out[...] = q1[0] / d + b2v[:, 128:256]


def _tc4_body(p0_ref, p1_ref, c_ref, out_ref):
    cnt = jnp.maximum((c_ref[0] + c_ref[1])[:, 0:1], 1.0)
    out_ref[...] = jnp.concatenate(
        [p0_ref[0] + p0_ref[1], p1_ref[0] + p1_ref[1]], axis=1) / cnt


def _sc_mesh():
    return plsc.VectorSubcoreMesh(core_axis_name="c", subcore_axis_name="s",
                                  num_cores=NC, num_subcores=NS)


def _lrelu_exp(e):
    return jnp.exp(jnp.maximum(e, 0.2 * e))


def _make_w_kernel(num_heads):
    hpc = 2 if num_heads == 4 else 1

    @functools.partial(
        pl.kernel,
        out_type=jax.ShapeDtypeStruct((num_heads, 1, E), jnp.float32),
        mesh=_sc_mesh(),
        compiler_params=pltpu.CompilerParams(needs_layout_passes=False),
        scratch_types=[pltpu.VMEM((N,), jnp.float32)] * (2 * hpc)
                      + [pltpu.VMEM((ECH,), jnp.int32)] * 2
                      + [pltpu.VMEM((ECH,), jnp.float32)] * hpc
                      + [pltpu.SemaphoreType.DMA],
    )
    def w_kernel(asT_h, adT_h, src_h, dst_h, w_out, *refs):
        a_vs = refs[:hpc]
        b_vs = refs[hpc:2 * hpc]
        ssrc, sdst = refs[2 * hpc:2 * hpc + 2]
        wbufs = refs[2 * hpc + 2:2 * hpc + 2 + hpc]
        sem = refs[-1]
        c = lax.axis_index("c")
        s = lax.axis_index("s")

        def tile_work():
            for hi in range(hpc):
                hsel = hpc * c + hi if num_heads == 4 else hi
                pltpu.sync_copy(asT_h.at[hsel, 0], a_vs[hi])
                pltpu.sync_copy(adT_h.at[hsel, 0], b_vs[hi])
            if num_heads == 4:
                wid, stride = s, NS
                nch = jnp.where(wid < NCHR, NCHF + 1, NCHF)
            else:
                wid, stride = c * NS + s, NW
                nfull = NCHG // NW
                nch = jnp.where(wid < NCHG - nfull * NW, nfull + 1, nfull)

            def chunk(k, _):
                cb = (wid + k * stride) * ECH
                c1 = pltpu.async_copy(src_h.at[pl.ds(cb, ECH)], ssrc, sem)
                c2 = pltpu.async_copy(dst_h.at[pl.ds(cb, ECH)], sdst, sem)
                c1.wait()
                c2.wait()
                for t in range(ECH // LANES):
                    sl = pl.ds(t * LANES, LANES)
                    s16 = ssrc[sl]
                    d16 = sdst[sl]
                    for hi in range(hpc):
                        e = (plsc.load_gather(a_vs[hi], [s16])
                             + plsc.load_gather(b_vs[hi], [d16]))
                        wbufs[hi][sl] = _lrelu_exp(e)
                for hi in range(hpc):
                    hsel = hpc * c + hi if num_heads == 4 else hi
                    pltpu.sync_copy(wbufs[hi],
                                    w_out.at[hsel, 0, pl.ds(cb, ECH)])
                return 0
            lax.fori_loop(0, nch, chunk, 0)

        tile_work()

    return w_kernel


def _make_pair_kernel():
    @functools.partial(
        pl.kernel,
        out_type=(jax.ShapeDtypeStruct((NC, N_PAD, HID), jnp.float32),
                  jax.ShapeDtypeStruct((NC, DEN_R, 128), jnp.float32)),
        mesh=_sc_mesh(),
        compiler_params=pltpu.CompilerParams(needs_layout_passes=False),
        scratch_types=[
            pltpu.VMEM((DEN_R, 128), jnp.float32),
            pltpu.VMEM((DEN_R,), jnp.int32),
            pltpu.VMEM((ECH,), jnp.int32),
            pltpu.VMEM((ECH,), jnp.int32),
            pltpu.VMEM((ECH,), jnp.float32),
        ]
        + [pltpu.VMEM((EB, HID), jnp.float32)] * 4
        + [pltpu.VMEM((EB,), jnp.int32)] * 4
        + [pltpu.VMEM_SHARED((N_PAD, HID), jnp.float32),
           pltpu.VMEM_SHARED((DEN_R, 128), jnp.float32)]
        + [pltpu.SemaphoreType.DMA] * 9,
    )
    def pair_kernel(src_h, dst_h, w_h, tbl_h, rid_h, out_num, out_den,
                    den_v, rid_v, ssrc, sdst, sw,
                    r0, r1, r2, r3, db0, db1, db2, db3,
                    num_sp, den_sp,
                    sem_st, g0, g1, g2, g3, s0, s1, s2, s3):
        rows = (r0, r1, r2, r3)
        dbs = (db0, db1, db2, db3)
        gsems = (g0, g1, g2, g3)
        ssems = (s0, s1, s2, s3)
        c = lax.axis_index("c")
        s = lax.axis_index("s")
        wline = w_h.at[c, 0]
        tline = tbl_h.at[c]

        pltpu.sync_copy(rid_h, rid_v)
        zero16 = jnp.zeros((LANES,), jnp.float32)

        def _zrow(r, _):
            for cc in range(HID // LANES):
                r0[r, pl.ds(cc * LANES, LANES)] = zero16
            return 0
        lax.fori_loop(0, EB, _zrow, 0)

        def _zden(r, _):
            for cc in range(128 // LANES):
                den_v[r, pl.ds(cc * LANES, LANES)] = zero16
            return 0
        lax.fori_loop(0, DEN_R, _zden, 0)

        nbase = s * NPT
        for k in range(NPT // EB):
            pltpu.sync_copy(r0, num_sp.at[pl.ds(nbase + k * EB, EB)])

        @pl.when(s < DEN_R // 8)
        def _():
            pltpu.sync_copy(den_v.at[pl.ds(0, 8)], den_sp.at[pl.ds(s * 8, 8)])
        plsc.subcore_barrier()

        def _den_update(idx_ref, woff, n16):
            for t in range(n16):
                sl = pl.ds(t * LANES, LANES)
                d16 = idx_ref[sl]
                plsc.addupdate_scatter(
                    den_v,
                    [lax.shift_right_logical(d16, 7),
                     lax.bitwise_and(d16, 127)],
                    sw[pl.ds(woff + t * LANES, LANES)])

        def _scale_rows(buf, woff, nrows):
            zi = jnp.zeros((LANES,), jnp.int32)

            def body(r2, _):
                r = r2 * 2
                wb0 = plsc.load_gather(sw, [zi + (woff + r)])
                wb1 = plsc.load_gather(sw, [zi + (woff + r + 1)])
                for cc in range(HID // LANES):
                    sl = pl.ds(cc * LANES, LANES)
                    buf[r, sl] = buf[r, sl] * wb0
                    buf[r + 1, sl] = buf[r + 1, sl] * wb1
                return 0
            lax.fori_loop(0, nrows // 2, body, 0)

        nch = jnp.where(s < NCHR, NCHF + 1, NCHF)

        def chunk(k, _):
            cb = (s + k * NS) * ECH
            c1 = pltpu.async_copy(src_h.at[pl.ds(cb, ECH)], ssrc, sem_st)
            c2 = pltpu.async_copy(dst_h.at[pl.ds(cb, ECH)], sdst, sem_st)
            c3 = pltpu.async_copy(wline.at[pl.ds(cb, ECH)], sw, sem_st)
            c1.wait()
            c2.wait()
            c3.wait()
            gdesc = {}
            sdesc = {}
            for b in (0, 1):
                gdesc[b] = pltpu.async_copy(
                    tline.at[ssrc.at[pl.ds(b * EB, EB)]], rows[b], gsems[b])
            for b in range(NBCH):
                nb = b + 2
                if nb < NBCH:
                    if nb - 4 >= 0:
                        sdesc[nb - 4].wait()
                    gdesc[nb] = pltpu.async_copy(
                        tline.at[ssrc.at[pl.ds(nb * EB, EB)]],
                        rows[nb % 4], gsems[nb % 4])
                gdesc[b].wait()
                for t in range(EB // LANES):
                    sl = pl.ds(t * LANES, LANES)
                    dbs[b % 4][sl] = sdst[pl.ds(b * EB + t * LANES, LANES)]
                _den_update(dbs[b % 4], b * EB, EB // LANES)
                _scale_rows(rows[b % 4], b * EB, EB)
                sdesc[b] = pltpu.async_copy(rows[b % 4],
                                            num_sp.at[dbs[b % 4]],
                                            ssems[b % 4], add=True)
            for b in range(NBCH - 4, NBCH):
                sdesc[b].wait()
            return 0
        lax.fori_loop(0, nch, chunk, 0)

        pltpu.sync_copy(den_v, den_sp.at[rid_v], add=True)
        plsc.subcore_barrier()

        for k in range(NPT // EB):
            pltpu.sync_copy(num_sp.at[pl.ds(nbase + k * EB, EB)],
                            out_num.at[c, pl.ds(nbase + k * EB, EB)])

        @pl.when(s < DEN_R // 8)
        def _():
            pltpu.sync_copy(den_sp.at[pl.ds(s * 8, 8)],
                            out_den.at[c, pl.ds(s * 8, 8)])

    return pair_kernel


def _make_pool_kernel():
    NBP = N // PB
    TAIL = N - NBP * PB

    @functools.partial(
        pl.kernel,
        out_type=(jax.ShapeDtypeStruct((NC, G, 128), jnp.float32),
                  jax.ShapeDtypeStruct((NC, G, 128), jnp.float32),
                  jax.ShapeDtypeStruct((NC, G, 128), jnp.float32)),
        mesh=_sc_mesh(),
        compiler_params=pltpu.CompilerParams(needs_layout_passes=False),
        scratch_types=[
            pltpu.VMEM((PB, 128), jnp.float32),
            pltpu.VMEM((PB, 128), jnp.float32),
            pltpu.VMEM((PB,), jnp.int32),
            pltpu.VMEM((TAIL, 128), jnp.float32),
            pltpu.VMEM((TAIL, 128), jnp.float32),
            pltpu.VMEM((TAIL,), jnp.int32),
            pltpu.VMEM((G, 128), jnp.float32),
            pltpu.VMEM((G,), jnp.int32),
            pltpu.VMEM_SHARED((G, 128), jnp.float32),
            pltpu.VMEM_SHARED((G, 128), jnp.float32),
            pltpu.VMEM_SHARED((G, 128), jnp.float32),
        ],
    )
    def pool_kernel(h0_hbm, h1_hbm, batch_hbm, gid_hbm,
                    out_p0, out_p1, out_cnt,
                    rows0_v, rows1_v, bid_v, trows0_v, trows1_v, tbid_v,
                    cnt_v, rid_v, p0_sp, p1_sp, cnt_sp):
        c = lax.axis_index("c")
        s = lax.axis_index("s")
        wid = c * NS + s

        pltpu.sync_copy(gid_hbm, rid_v)
        zero16 = jnp.zeros((LANES,), jnp.float32)
        one16 = jnp.ones((LANES,), jnp.float32)

        def _zcnt(r, _):
            for cc in range(128 // LANES):
                cnt_v[r, pl.ds(cc * LANES, LANES)] = zero16
                rows0_v[r, pl.ds(cc * LANES, LANES)] = zero16
            return 0
        lax.fori_loop(0, G, _zcnt, 0)

        @pl.when(s < G // 8)
        def _():
            pltpu.sync_copy(rows0_v.at[pl.ds(0, 8)], p0_sp.at[pl.ds(s * 8, 8)])
            pltpu.sync_copy(rows0_v.at[pl.ds(0, 8)], p1_sp.at[pl.ds(s * 8, 8)])
            pltpu.sync_copy(cnt_v.at[pl.ds(0, 8)], cnt_sp.at[pl.ds(s * 8, 8)])
        plsc.subcore_barrier()

        nfull = NBP // NW
        nb = jnp.where(wid < NBP - nfull * NW, nfull + 1, nfull)

        def pb(j, _):
            base = (wid + j * NW) * PB
            pltpu.sync_copy(h0_hbm.at[pl.ds(base, PB)], rows0_v)
            pltpu.sync_copy(h1_hbm.at[pl.ds(base, PB)], rows1_v)
            pltpu.sync_copy(batch_hbm.at[pl.ds(base, PB)], bid_v)
            for t in range(PB // LANES):
                b16 = bid_v[pl.ds(t * LANES, LANES)]
                plsc.addupdate_scatter(
                    cnt_v, [b16, jnp.zeros((LANES,), jnp.int32)], one16)
            pltpu.sync_copy(rows0_v, p0_sp.at[bid_v], add=True)
            pltpu.sync_copy(rows1_v, p1_sp.at[bid_v], add=True)
            return 0
        lax.fori_loop(0, nb, pb, 0)

        @pl.when(wid == NW - 1)
        def _():
            pltpu.sync_copy(h0_hbm.at[pl.ds(N - TAIL, TAIL)], trows0_v)
            pltpu.sync_copy(h1_hbm.at[pl.ds(N - TAIL, TAIL)], trows1_v)
            pltpu.sync_copy(batch_hbm.at[pl.ds(N - TAIL, TAIL)], tbid_v)
            t16 = tbid_v[pl.ds(0, LANES)]
            plsc.addupdate_scatter(
                cnt_v, [t16, jnp.zeros((LANES,), jnp.int32)], one16)
            pltpu.sync_copy(trows0_v, p0_sp.at[tbid_v], add=True)
            pltpu.sync_copy(trows1_v, p1_sp.at[tbid_v], add=True)

        pltpu.sync_copy(cnt_v, cnt_sp.at[rid_v], add=True)
        plsc.subcore_barrier()

        @pl.when(s < G // 8)
        def _():
            pltpu.sync_copy(p0_sp.at[pl.ds(s * 8, 8)],
                            out_p0.at[c, pl.ds(s * 8, 8)])
            pltpu.sync_copy(p1_sp.at[pl.ds(s * 8, 8)],
                            out_p1.at[c, pl.ds(s * 8, 8)])
            pltpu.sync_copy(cnt_sp.at[pl.ds(s * 8, 8)],
                            out_cnt.at[c, pl.ds(s * 8, 8)])

    return pool_kernel


def kernel(x, edge_index, batch, W1, att_src1, att_dst1, b1,
           W2, att_src2, att_dst2, b2):
    x = x.astype(jnp.float32)
    src = edge_index[0]
    dst = edge_index[1]
    rowids = jnp.arange(DEN_R, dtype=jnp.int32)

    eyeH = jnp.eye(HEADS, dtype=jnp.float32)
    As1 = jnp.einsum("hd,hg->hdg", att_src1, eyeH).reshape(HEADS * HID, HEADS)
    Ad1 = jnp.einsum("hd,hg->hdg", att_dst1, eyeH).reshape(HEADS * HID, HEADS)
    As2 = att_src2.reshape(D_OUT, 1)
    Ad2 = att_dst2.reshape(D_OUT, 1)

    grid = (N // BM,)
    row_spec = lambda w: pl.BlockSpec((BM, w), lambda i: (i, 0))
    pair_spec = pl.BlockSpec((NC, BM, HID), lambda i: (0, i, 0))
    full_spec = lambda a, b_: pl.BlockSpec((a, b_), lambda i: (0, 0))
    t01, t23, as1, ad1 = pl.pallas_call(
        _tc1_body,
        grid=grid,
        in_specs=[row_spec(D_IN), full_spec(D_IN, HEADS * HID),
                  full_spec(HEADS * HID, HEADS), full_spec(HEADS * HID, HEADS)],
        out_specs=[pair_spec] * 2 + [row_spec(HEADS)] * 2,
        out_shape=[jax.ShapeDtypeStruct((NC, N, HID), jnp.float32)] * 2
                  + [jax.ShapeDtypeStruct((N, HEADS), jnp.float32)] * 2,
    )(x, W1, As1, Ad1)

    w_k4 = _make_w_kernel(HEADS)
    w1 = w_k4(as1.T.reshape(HEADS, 1, N), ad1.T.reshape(HEADS, 1, N),
              src, dst)

    pair_k = _make_pair_kernel()
    n01, d01 = pair_k(src, dst, w1[0:2], t01, rowids)
    n23, d23 = pair_k(src, dst, w1[2:4], t23, rowids)
    den1 = jnp.stack([d01[0].reshape(N_PAD), d01[1].reshape(N_PAD),
                      d23[0].reshape(N_PAD), d23[1].reshape(N_PAD)], axis=-1)

    plane0 = pl.BlockSpec((1, BM, HID), lambda i: (0, i, 0))
    plane1 = pl.BlockSpec((1, BM, HID), lambda i: (1, i, 0))
    t2p, as2, ad2 = pl.pallas_call(
        _tc2_body,
        grid=grid,
        in_specs=[plane0, plane1, plane0, plane1,
                  pl.BlockSpec((BM, HEADS), lambda i: (i, 0)),
                  full_spec(1, HEADS * HID),
                  full_spec(HEADS * HID, D_OUT),
                  full_spec(D_OUT, 1), full_spec(D_OUT, 1)],
        out_specs=[pair_spec, row_spec(1), row_spec(1)],
        out_shape=[jax.ShapeDtypeStruct((NC, N, HID), jnp.float32),
                   jax.ShapeDtypeStruct((N, 1), jnp.float32),
                   jax.ShapeDtypeStruct((N, 1), jnp.float32)],
    )(n01, n01, n23, n23, den1, b1.reshape(1, HEADS * HID), W2, As2, Ad2)

    w_k1 = _make_w_kernel(1)
    w2 = w_k1(as2.reshape(1, 1, N), ad2.reshape(1, 1, N), src, dst)
    w2p = jnp.concatenate([w2, w2], axis=0)

    q01, dq = pair_k(src, dst, w2p, t2p, rowids)
    den2 = dq[0].reshape(N_PAD)[:, None]

    h0, h1 = pl.pallas_call(
        _tc3_body,
        grid=grid,
        in_specs=[plane0, plane1,
                  pl.BlockSpec((BM, 1), lambda i: (i, 0)),
                  full_spec(1, D_OUT)],
        out_specs=[row_spec(HID)] * 2,
        out_shape=[jax.ShapeDtypeStruct((N, HID), jnp.float32)] * 2,
    )(q01, q01, den2, b2.reshape(1, D_OUT))

    pool_k = _make_pool_kernel()
    gids = jnp.arange(G, dtype=jnp.int32)
    pool_p0, pool_p1, cnt_part = pool_k(h0, h1, batch, gids)

    pooled = pl.pallas_call(
        _tc4_body,
        grid=(1,),
        in_specs=[pl.BlockSpec((NC, G, 128), lambda i: (0, 0, 0))] * 3,
        out_specs=pl.BlockSpec((G, D_OUT), lambda i: (0, 0)),
        out_shape=jax.ShapeDtypeStruct((G, D_OUT), jnp.float32),
    )(pool_p0, pool_p1, cnt_part)
    return pooled
